# baseline (device time: 33857 ns/iter reference)
import jax
import jax.numpy as jnp
from jax import lax
from jax.experimental import pallas as pl
from jax.experimental.pallas import tpu as pltpu

N_DEV = 8
CH = 128

X, Y, Z = 1, 3, 4
ORDERS = ((X, Y, Z), (Y, Z, X), (Z, X, Y))
STRIPES = ((0, 384), (384, 384), (768, 256))


def _span(masks):
    out = [0]
    for m in masks:
        out = out + [v ^ m for v in out]
    return out


def kernel(A, B):
    m, _ = A.shape
    _, n = B.shape

    def body(a_ref, b_ref, out_ref, a_v, b_v, o_v, part_ref,
             rbuf0, rbuf1, rbuf2, sbuf0, sbuf1, sbuf2,
             gbuf0, gbuf1, gbuf2,
             rs_send, rs_recv, ag_send, ag_recv, in_sems, out_sems):
        rbufs = (rbuf0, rbuf1, rbuf2)
        sbufs = (sbuf0, sbuf1, sbuf2)
        gbufs = (gbuf0, gbuf1, gbuf2)
        my = lax.axis_index("i")
        all_rdmas = []
        all_outcopies = []

        a_copy = pltpu.make_async_copy(a_ref, a_v, in_sems.at[0])
        a_copy.start()
        b_copies = []
        for r in range(3):
            c0, w = STRIPES[r]
            cols = pl.ds(c0, w)
            bc = pltpu.make_async_copy(
                b_ref.at[:, cols], b_v.at[:, cols], in_sems.at[1 + r]
            )
            bc.start()
            b_copies.append(bc)

        def store_out(r, slot, rows, cols):
            oc = pltpu.make_async_copy(
                o_v.at[rows, cols], out_ref.at[rows, cols],
                out_sems.at[r, slot],
            )
            oc.start()
            all_outcopies.append(oc)

        barrier_sem = pltpu.get_barrier_semaphore()
        for mask in (X, Y, Z):
            pl.semaphore_signal(
                barrier_sem, inc=1,
                device_id=(my ^ mask,), device_id_type=pl.DeviceIdType.MESH,
            )

        def issue_rs(r, t, prev_slot_by_s, prestaged=False):
            order = ORDERS[r]
            mask = order[t]
            rest = _span(order[t + 1:])
            c0, w = STRIPES[r]
            slot0 = 8 - 2 * len(rest)
            out = []
            for k, s in enumerate(rest):
                slot = slot0 + k
                c_send = my ^ mask ^ s
                if not prestaged:
                    rows = pl.ds(c_send * CH, CH)
                    val = part_ref[rows, pl.ds(c0, w)]
                    if t > 0:
                        pslot = prev_slot_by_s[mask ^ s]
                        val = val + rbufs[r][pslot, :, :].astype(jnp.float32)
                    sbufs[r][slot, :, :] = val.astype(jnp.bfloat16)
                rdma = pltpu.make_async_remote_copy(
                    src_ref=sbufs[r].at[slot],
                    dst_ref=rbufs[r].at[slot],
                    send_sem=rs_send.at[r, slot],
                    recv_sem=rs_recv.at[r, slot],
                    device_id=(my ^ mask,),
                    device_id_type=pl.DeviceIdType.MESH,
                )
                rdma.start()
                all_rdmas.append(rdma)
                out.append((rdma, slot, s))
            return out

        def ag_plan(r):
            ag_order = ORDERS[r][::-1]
            sends = {}
            recv_label = {}
            for t in range(3):
                have = _span(ag_order[:t])
                slot0 = len(have) - 1
                for k, s in enumerate(have):
                    sends.setdefault(s, []).append(
                        (slot0 + k, ag_order[t])
                    )
                    recv_label[slot0 + k] = ag_order[t] ^ s
            return sends, recv_label

        ag_pend = {r: {} for r in range(3)}

        def issue_ag_chunk(r, lam):
            sends, _ = ag_plan(r)
            c_send = my ^ lam
            for slot, mask in sends.get(lam, []):
                rdma = pltpu.make_async_remote_copy(
                    src_ref=gbufs[r].at[c_send],
                    dst_ref=gbufs[r].at[c_send],
                    send_sem=ag_send.at[r, slot],
                    recv_sem=ag_recv.at[r, slot],
                    device_id=(my ^ mask,),
                    device_id_type=pl.DeviceIdType.MESH,
                )
                rdma.start()
                all_rdmas.append(rdma)
                ag_pend[r][slot] = rdma

        a_copy.wait()
        for r in range(3):
            order = ORDERS[r]
            c0, w = STRIPES[r]
            b_copies[r].wait()
            part_ref[:, pl.ds(c0, w)] = jnp.dot(
                a_v[...], b_v[:, pl.ds(c0, w)],
                preferred_element_type=jnp.float32,
            )
            for k, s in enumerate(_span(order[1:])):
                c_send = my ^ order[0] ^ s
                sbufs[r][k, :, :] = part_ref[
                    pl.ds(c_send * CH, CH), pl.ds(c0, w)
                ].astype(jnp.bfloat16)
        pl.semaphore_wait(barrier_sem, 3)
        pending = {r: issue_rs(r, 0, None, prestaged=True) for r in range(3)}

        for t in range(3):
            for r in range(3):
                order = ORDERS[r]
                c0, w = STRIPES[r]
                cur = pending[r]
                slot_by_s = {s: slot for (_, slot, s) in cur}
                if t < 2:
                    fwd = {order[t + 1] ^ s2 for s2 in _span(order[t + 2:])}
                    for rdma, slot, s in cur:
                        if s in fwd:
                            rdma.wait_recv()
                    pending[r] = issue_rs(r, t + 1, slot_by_s)
                    for rdma, slot, s in cur:
                        if s not in fwd:
                            rdma.wait_recv()
                            rows = pl.ds((my ^ s) * CH, CH)
                            part_ref[rows, pl.ds(c0, w)] = (
                                part_ref[rows, pl.ds(c0, w)]
                                + rbufs[r][slot, :, :].astype(jnp.float32)
                            )
                else:
                    rdma, slot, s = cur[0]
                    rdma.wait_recv()
                    rows = pl.ds(my * CH, CH)
                    z = (
                        part_ref[rows, pl.ds(c0, w)]
                        + rbufs[r][slot, :, :].astype(jnp.float32)
                    )
                    act = z * jax.nn.sigmoid(z)
                    o_v[rows, pl.ds(c0, w)] = act
                    gbufs[r][pl.ds(my, 1), :, :] = act.astype(jnp.bfloat16)[
                        None, :, :
                    ]
                    issue_ag_chunk(r, 0)
                    store_out(r, 7, rows, pl.ds(c0, w))

        for slot in range(7):
            for r in range(3):
                c0, w = STRIPES[r]
                _, recv_label = ag_plan(r)
                lam = recv_label[slot]
                ag_pend[r][slot].wait_recv()
                issue_ag_chunk(r, lam)
                c = my ^ lam
                o_v[pl.ds(c * CH, CH), pl.ds(c0, w)] = gbufs[r][
                    c, :, :
                ].astype(jnp.float32)
                store_out(r, slot, pl.ds(c * CH, CH), pl.ds(c0, w))

        for oc in all_outcopies:
            oc.wait()
        for rdma in all_rdmas:
            rdma.wait_send()

    return pl.pallas_call(
        body,
        out_shape=jax.ShapeDtypeStruct((m, n), jnp.float32),
        in_specs=[
            pl.BlockSpec(memory_space=pl.ANY),
            pl.BlockSpec(memory_space=pl.ANY),
        ],
        out_specs=pl.BlockSpec(memory_space=pl.ANY),
        scratch_shapes=[
            pltpu.VMEM(A.shape, jnp.float32),
            pltpu.VMEM(B.shape, jnp.float32),
            pltpu.VMEM((m, n), jnp.float32),
            pltpu.VMEM((m, n), jnp.float32),
            pltpu.VMEM((7, CH, 384), jnp.bfloat16),
            pltpu.VMEM((7, CH, 384), jnp.bfloat16),
            pltpu.VMEM((7, CH, 256), jnp.bfloat16),
            pltpu.VMEM((7, CH, 384), jnp.bfloat16),
            pltpu.VMEM((7, CH, 384), jnp.bfloat16),
            pltpu.VMEM((7, CH, 256), jnp.bfloat16),
            pltpu.VMEM((N_DEV, CH, 384), jnp.bfloat16),
            pltpu.VMEM((N_DEV, CH, 384), jnp.bfloat16),
            pltpu.VMEM((N_DEV, CH, 256), jnp.bfloat16),
            pltpu.SemaphoreType.DMA((3, 7)),
            pltpu.SemaphoreType.DMA((3, 7)),
            pltpu.SemaphoreType.DMA((3, 7)),
            pltpu.SemaphoreType.DMA((3, 7)),
            pltpu.SemaphoreType.DMA((4,)),
            pltpu.SemaphoreType.DMA((3, 8)),
        ],
        compiler_params=pltpu.CompilerParams(collective_id=0),
    )(A, B)


# device time: 32144 ns/iter; 1.0533x vs baseline; 1.0533x over previous
import jax
import jax.numpy as jnp
from jax import lax
from jax.experimental import pallas as pl
from jax.experimental.pallas import tpu as pltpu

N_DEV = 8
CH = 128

X, Y, Z = 1, 3, 4
ORDERS = ((X, Y, Z), (Y, Z, X), (Z, X, Y))
STRIPES = ((0, 384), (384, 384), (768, 256))


def _span(masks):
    out = [0]
    for m in masks:
        out = out + [v ^ m for v in out]
    return out


def kernel(A, B):
    m, _ = A.shape
    _, n = B.shape

    def body(a_ref, b_ref, out_ref, part_ref,
             rbuf0, rbuf1, rbuf2, sbuf0, sbuf1, sbuf2,
             gbuf0, gbuf1, gbuf2,
             rs_send, rs_recv, ag_send, ag_recv):
        rbufs = (rbuf0, rbuf1, rbuf2)
        sbufs = (sbuf0, sbuf1, sbuf2)
        gbufs = (gbuf0, gbuf1, gbuf2)
        my = lax.axis_index("i")
        all_rdmas = []

        barrier_sem = pltpu.get_barrier_semaphore()
        for mask in (X, Y, Z):
            pl.semaphore_signal(
                barrier_sem, inc=1,
                device_id=(my ^ mask,), device_id_type=pl.DeviceIdType.MESH,
            )

        def issue_rs(r, t, prev_slot_by_s, prestaged=False):
            order = ORDERS[r]
            mask = order[t]
            rest = _span(order[t + 1:])
            c0, w = STRIPES[r]
            slot0 = 8 - 2 * len(rest)
            out = []
            for k, s in enumerate(rest):
                slot = slot0 + k
                c_send = my ^ mask ^ s
                if not prestaged:
                    rows = pl.ds(c_send * CH, CH)
                    val = part_ref[rows, pl.ds(c0, w)]
                    if t > 0:
                        pslot = prev_slot_by_s[mask ^ s]
                        val = val + rbufs[r][pslot, :, :].astype(jnp.float32)
                    sbufs[r][slot, :, :] = val.astype(jnp.bfloat16)
                rdma = pltpu.make_async_remote_copy(
                    src_ref=sbufs[r].at[slot],
                    dst_ref=rbufs[r].at[slot],
                    send_sem=rs_send.at[r, slot],
                    recv_sem=rs_recv.at[r, slot],
                    device_id=(my ^ mask,),
                    device_id_type=pl.DeviceIdType.MESH,
                )
                rdma.start()
                all_rdmas.append(rdma)
                out.append((rdma, slot, s))
            return out

        def ag_plan(r):
            ag_order = ORDERS[r][::-1]
            sends = {}
            recv_label = {}
            for t in range(3):
                have = _span(ag_order[:t])
                slot0 = len(have) - 1
                for k, s in enumerate(have):
                    sends.setdefault(s, []).append(
                        (slot0 + k, ag_order[t])
                    )
                    recv_label[slot0 + k] = ag_order[t] ^ s
            return sends, recv_label

        ag_pend = {r: {} for r in range(3)}

        def issue_ag_chunk(r, lam):
            sends, _ = ag_plan(r)
            c_send = my ^ lam
            for slot, mask in sends.get(lam, []):
                rdma = pltpu.make_async_remote_copy(
                    src_ref=gbufs[r].at[c_send],
                    dst_ref=gbufs[r].at[c_send],
                    send_sem=ag_send.at[r, slot],
                    recv_sem=ag_recv.at[r, slot],
                    device_id=(my ^ mask,),
                    device_id_type=pl.DeviceIdType.MESH,
                )
                rdma.start()
                all_rdmas.append(rdma)
                ag_pend[r][slot] = rdma

        for r in range(3):
            order = ORDERS[r]
            c0, w = STRIPES[r]
            part_ref[:, pl.ds(c0, w)] = jnp.dot(
                a_ref[...], b_ref[:, pl.ds(c0, w)],
                preferred_element_type=jnp.float32,
            )
            for k, s in enumerate(_span(order[1:])):
                c_send = my ^ order[0] ^ s
                sbufs[r][k, :, :] = part_ref[
                    pl.ds(c_send * CH, CH), pl.ds(c0, w)
                ].astype(jnp.bfloat16)
        pl.semaphore_wait(barrier_sem, 3)
        pending = {r: issue_rs(r, 0, None, prestaged=True) for r in range(3)}

        for t in range(3):
            for r in range(3):
                order = ORDERS[r]
                c0, w = STRIPES[r]
                cur = pending[r]
                slot_by_s = {s: slot for (_, slot, s) in cur}
                if t < 2:
                    fwd = {order[t + 1] ^ s2 for s2 in _span(order[t + 2:])}
                    for rdma, slot, s in cur:
                        if s in fwd:
                            rdma.wait_recv()
                    pending[r] = issue_rs(r, t + 1, slot_by_s)
                    for rdma, slot, s in cur:
                        if s not in fwd:
                            rdma.wait_recv()
                            rows = pl.ds((my ^ s) * CH, CH)
                            part_ref[rows, pl.ds(c0, w)] = (
                                part_ref[rows, pl.ds(c0, w)]
                                + rbufs[r][slot, :, :].astype(jnp.float32)
                            )
                else:
                    rdma, slot, s = cur[0]
                    rdma.wait_recv()
                    rows = pl.ds(my * CH, CH)
                    z = (
                        part_ref[rows, pl.ds(c0, w)]
                        + rbufs[r][slot, :, :].astype(jnp.float32)
                    )
                    act = z * jax.nn.sigmoid(z)
                    out_ref[rows, pl.ds(c0, w)] = act
                    gbufs[r][pl.ds(my, 1), :, :] = act.astype(jnp.bfloat16)[
                        None, :, :
                    ]
                    issue_ag_chunk(r, 0)

        for slot in range(7):
            for r in range(3):
                c0, w = STRIPES[r]
                _, recv_label = ag_plan(r)
                lam = recv_label[slot]
                ag_pend[r][slot].wait_recv()
                issue_ag_chunk(r, lam)
                c = my ^ lam
                out_ref[pl.ds(c * CH, CH), pl.ds(c0, w)] = gbufs[r][
                    c, :, :
                ].astype(jnp.float32)

        for rdma in all_rdmas:
            rdma.wait_send()

    return pl.pallas_call(
        body,
        out_shape=jax.ShapeDtypeStruct((m, n), jnp.float32),
        in_specs=[
            pl.BlockSpec(memory_space=pltpu.VMEM),
            pl.BlockSpec(memory_space=pltpu.VMEM),
        ],
        out_specs=pl.BlockSpec(memory_space=pltpu.VMEM),
        scratch_shapes=[
            pltpu.VMEM((m, n), jnp.float32),
            pltpu.VMEM((7, CH, 384), jnp.bfloat16),
            pltpu.VMEM((7, CH, 384), jnp.bfloat16),
            pltpu.VMEM((7, CH, 256), jnp.bfloat16),
            pltpu.VMEM((7, CH, 384), jnp.bfloat16),
            pltpu.VMEM((7, CH, 384), jnp.bfloat16),
            pltpu.VMEM((7, CH, 256), jnp.bfloat16),
            pltpu.VMEM((N_DEV, CH, 384), jnp.bfloat16),
            pltpu.VMEM((N_DEV, CH, 384), jnp.bfloat16),
            pltpu.VMEM((N_DEV, CH, 256), jnp.bfloat16),
            pltpu.SemaphoreType.DMA((3, 7)),
            pltpu.SemaphoreType.DMA((3, 7)),
            pltpu.SemaphoreType.DMA((3, 7)),
            pltpu.SemaphoreType.DMA((3, 7)),
        ],
        compiler_params=pltpu.CompilerParams(collective_id=0),
    )(A, B)


# device time: 32120 ns/iter; 1.0541x vs baseline; 1.0007x over previous
import jax
import jax.numpy as jnp
from jax import lax
from jax.experimental import pallas as pl
from jax.experimental.pallas import tpu as pltpu

N_DEV = 8
CH = 128

X, Y, Z = 1, 3, 4
ORDERS = ((X, Y, Z), (Y, Z, X), (Z, X, Y))
STRIPES = ((0, 384), (384, 384), (768, 256))


def _span(masks):
    out = [0]
    for m in masks:
        out = out + [v ^ m for v in out]
    return out


def kernel(A, B):
    m, _ = A.shape
    _, n = B.shape

    def body(a_ref, b_ref, out_ref, part_ref,
             rbuf0, rbuf1, rbuf2, sbuf0, sbuf1, sbuf2,
             gbuf0, gbuf1, gbuf2,
             rs_send, rs_recv, ag_send, ag_recv):
        rbufs = (rbuf0, rbuf1, rbuf2)
        sbufs = (sbuf0, sbuf1, sbuf2)
        gbufs = (gbuf0, gbuf1, gbuf2)
        my = lax.axis_index("i")
        all_rdmas = []

        barrier_sem = pltpu.get_barrier_semaphore()
        for mask in (X, Y, Z):
            pl.semaphore_signal(
                barrier_sem, inc=1,
                device_id=(my ^ mask,), device_id_type=pl.DeviceIdType.MESH,
            )

        def issue_rs(r, t, prev_slot_by_s, prestaged=False):
            order = ORDERS[r]
            mask = order[t]
            rest = _span(order[t + 1:])
            c0, w = STRIPES[r]
            slot0 = 8 - 2 * len(rest)
            out = []
            for k, s in enumerate(rest):
                slot = slot0 + k
                c_send = my ^ mask ^ s
                if not prestaged:
                    rows = pl.ds(c_send * CH, CH)
                    val = part_ref[rows, pl.ds(c0, w)]
                    if t > 0:
                        pslot = prev_slot_by_s[mask ^ s]
                        val = val + rbufs[r][pslot, :, :].astype(jnp.float32)
                    sbufs[r][slot, :, :] = val.astype(jnp.bfloat16)
                rdma = pltpu.make_async_remote_copy(
                    src_ref=sbufs[r].at[slot],
                    dst_ref=rbufs[r].at[slot],
                    send_sem=rs_send.at[r, slot],
                    recv_sem=rs_recv.at[r, slot],
                    device_id=(my ^ mask,),
                    device_id_type=pl.DeviceIdType.MESH,
                )
                rdma.start()
                all_rdmas.append(rdma)
                out.append((rdma, slot, s))
            return out

        def ag_plan(r):
            ag_order = ORDERS[r][::-1]
            sends = {}
            recv_label = {}
            for t in range(3):
                have = _span(ag_order[:t])
                slot0 = len(have) - 1
                for k, s in enumerate(have):
                    sends.setdefault(s, []).append(
                        (slot0 + k, ag_order[t])
                    )
                    recv_label[slot0 + k] = ag_order[t] ^ s
            return sends, recv_label

        ag_pend = {r: {} for r in range(3)}

        def issue_ag_chunk(r, lam):
            sends, _ = ag_plan(r)
            c_send = my ^ lam
            for slot, mask in sends.get(lam, []):
                rdma = pltpu.make_async_remote_copy(
                    src_ref=gbufs[r].at[c_send],
                    dst_ref=gbufs[r].at[c_send],
                    send_sem=ag_send.at[r, slot],
                    recv_sem=ag_recv.at[r, slot],
                    device_id=(my ^ mask,),
                    device_id_type=pl.DeviceIdType.MESH,
                )
                rdma.start()
                all_rdmas.append(rdma)
                ag_pend[r][slot] = rdma

        for r in range(3):
            order = ORDERS[r]
            c0, w = STRIPES[r]
            part_ref[:, pl.ds(c0, w)] = jnp.dot(
                a_ref[...], b_ref[:, pl.ds(c0, w)],
                preferred_element_type=jnp.float32,
            )
            for k, s in enumerate(_span(order[1:])):
                c_send = my ^ order[0] ^ s
                sbufs[r][k, :, :] = part_ref[
                    pl.ds(c_send * CH, CH), pl.ds(c0, w)
                ].astype(jnp.bfloat16)
        pl.semaphore_wait(barrier_sem, 3)
        pending = {r: issue_rs(r, 0, None, prestaged=True) for r in range(3)}

        for t in range(3):
            deferred = []
            for r in range(3):
                order = ORDERS[r]
                c0, w = STRIPES[r]
                cur = pending[r]
                slot_by_s = {s: slot for (_, slot, s) in cur}
                if t < 2:
                    fwd = {order[t + 1] ^ s2 for s2 in _span(order[t + 2:])}
                    for rdma, slot, s in cur:
                        if s in fwd:
                            rdma.wait_recv()
                    pending[r] = issue_rs(r, t + 1, slot_by_s)
                    deferred.extend(
                        (r, rdma, slot, s)
                        for rdma, slot, s in cur
                        if s not in fwd
                    )
                else:
                    rdma, slot, s = cur[0]
                    rdma.wait_recv()
                    rows = pl.ds(my * CH, CH)
                    z = (
                        part_ref[rows, pl.ds(c0, w)]
                        + rbufs[r][slot, :, :].astype(jnp.float32)
                    )
                    act = z * jax.nn.sigmoid(z)
                    out_ref[rows, pl.ds(c0, w)] = act
                    gbufs[r][pl.ds(my, 1), :, :] = act.astype(jnp.bfloat16)[
                        None, :, :
                    ]
                    issue_ag_chunk(r, 0)
            for r, rdma, slot, s in deferred:
                c0, w = STRIPES[r]
                rdma.wait_recv()
                rows = pl.ds((my ^ s) * CH, CH)
                part_ref[rows, pl.ds(c0, w)] = (
                    part_ref[rows, pl.ds(c0, w)]
                    + rbufs[r][slot, :, :].astype(jnp.float32)
                )

        for slot in range(7):
            for r in range(3):
                c0, w = STRIPES[r]
                _, recv_label = ag_plan(r)
                lam = recv_label[slot]
                ag_pend[r][slot].wait_recv()
                issue_ag_chunk(r, lam)
                c = my ^ lam
                out_ref[pl.ds(c * CH, CH), pl.ds(c0, w)] = gbufs[r][
                    c, :, :
                ].astype(jnp.float32)

        for rdma in all_rdmas:
            rdma.wait_send()

    return pl.pallas_call(
        body,
        out_shape=jax.ShapeDtypeStruct((m, n), jnp.float32),
        in_specs=[
            pl.BlockSpec(memory_space=pltpu.VMEM),
            pl.BlockSpec(memory_space=pltpu.VMEM),
        ],
        out_specs=pl.BlockSpec(memory_space=pltpu.VMEM),
        scratch_shapes=[
            pltpu.VMEM((m, n), jnp.float32),
            pltpu.VMEM((7, CH, 384), jnp.bfloat16),
            pltpu.VMEM((7, CH, 384), jnp.bfloat16),
            pltpu.VMEM((7, CH, 256), jnp.bfloat16),
            pltpu.VMEM((7, CH, 384), jnp.bfloat16),
            pltpu.VMEM((7, CH, 384), jnp.bfloat16),
            pltpu.VMEM((7, CH, 256), jnp.bfloat16),
            pltpu.VMEM((N_DEV, CH, 384), jnp.bfloat16),
            pltpu.VMEM((N_DEV, CH, 384), jnp.bfloat16),
            pltpu.VMEM((N_DEV, CH, 256), jnp.bfloat16),
            pltpu.SemaphoreType.DMA((3, 7)),
            pltpu.SemaphoreType.DMA((3, 7)),
            pltpu.SemaphoreType.DMA((3, 7)),
            pltpu.SemaphoreType.DMA((3, 7)),
        ],
        compiler_params=pltpu.CompilerParams(collective_id=0),
    )(A, B)
